# baseline (speedup 1.0000x reference)
import functools

import jax
import jax.numpy as jnp
from jax import lax
from jax.experimental import pallas as pl
from jax.experimental.pallas import tpu as pltpu
from jax.experimental.pallas import tpu_sc as plsc

N = 10000
D = 128
H = 128
E = 320000
L = 3

NC = 2
NS = 16
NW = NC * NS
EPW = E // NW
CH = 80
NCHUNK = EPW // CH
RING = 4
NROUND = NCHUNK // RING
RPT = N // NS
NP = 10240
RPTP = NP // NS
ZR = 128
HB = 640
CHI = 2000
EPC = E // NC

RB = 2000
G = N // RB


def _sc_scatter_body(y_hbm, idx_hbm, z_hbm, out_hbm, ib, rows, acc, *sems):
    gsx = sems[0:RING]
    ssx = sems[RING:2 * RING]
    isx = sems[2 * RING:3 * RING]
    c = lax.axis_index("c")
    s = lax.axis_index("s")
    w = c * NS + s

    pltpu.sync_copy(z_hbm, acc.at[pl.ds(s * RPTP, RPTP)])
    plsc.subcore_barrier()

    base = w * NCHUNK

    for b in range(RING):
        pltpu.async_copy(idx_hbm.at[base + b], ib.at[0, b], isx[b])

    def _round(r, carry):
        par = r % 2
        q0 = base + RING * r
        for b in range(RING):
            @pl.when(r > 0)
            def _():
                pltpu.make_async_copy(
                    y_hbm.at[pl.ds(0, CH)], rows.at[b], ssx[b]).wait()
            pltpu.make_async_copy(
                idx_hbm.at[q0 + b], ib.at[par, b], isx[b]).wait()
            pltpu.async_copy(y_hbm.at[ib.at[par, b, 0]], rows.at[b], gsx[b])
        for b in range(RING):
            pltpu.make_async_copy(
                y_hbm.at[pl.ds(0, CH)], rows.at[b], gsx[b]).wait()
            pltpu.async_copy(rows.at[b], acc.at[ib.at[par, b, 1]], ssx[b],
                             add=True)

            @pl.when(r < NROUND - 1)
            def _():
                pltpu.async_copy(idx_hbm.at[q0 + RING + b],
                                 ib.at[1 - par, b], isx[b])
        return carry
    lax.fori_loop(0, NROUND, _round, 0)

    for b in range(RING):
        pltpu.make_async_copy(
            y_hbm.at[pl.ds(0, CH)], rows.at[b], ssx[b]).wait()

    pltpu.sync_copy(idx_hbm.at[base + NROUND * RING], ib.at[0, 0])
    pltpu.async_copy(y_hbm.at[ib.at[0, 0, 0]], rows.at[0], gsx[0]).wait()
    pltpu.sync_copy(rows.at[0], acc.at[ib.at[0, 0, 1]], add=True)

    plsc.subcore_barrier()
    pltpu.sync_copy(acc.at[pl.ds(s * RPTP, RPTP)],
                    out_hbm.at[pl.ds(c * NP + s * RPTP, RPTP)])


@functools.lru_cache(maxsize=None)
def _get_sc_scatter():
    return pl.kernel(
        _sc_scatter_body,
        out_type=jax.ShapeDtypeStruct((NC * NP, D), jnp.float32),
        mesh=plsc.VectorSubcoreMesh(core_axis_name="c", subcore_axis_name="s"),
        compiler_params=pltpu.CompilerParams(needs_layout_passes=False),
        scratch_types=[
            pltpu.VMEM((2, RING, 2, CH), jnp.int32),
            pltpu.VMEM((RING, CH, D), jnp.float32),
            pltpu.VMEM_SHARED((NP, D), jnp.float32),
        ] + [pltpu.SemaphoreType.DMA] * (3 * RING),
    )


def _sc_scatter(y, pidx, zrows):
    return _get_sc_scatter()(y, pidx, zrows).reshape(NC, NP, D)


def _sc_hist_body(dst_hbm, out_hbm, didx, ones_v, zb, dacc, *sems):
    ssx = sems[0:RING]
    isx = sems[RING:2 * RING]
    c = lax.axis_index("c")
    s = lax.axis_index("s")
    w = c * NS + s

    def _fill(i, carry):
        ones_v[pl.ds(i * 16, 16)] = jnp.ones((16,), jnp.float32)
        return carry
    lax.fori_loop(0, CH // 16, _fill, 0)

    def _zb(i, carry):
        zb[pl.ds(i * 16, 16)] = jnp.zeros((16,), jnp.float32)
        return carry
    lax.fori_loop(0, RPTP // 16, _zb, 0)
    pltpu.sync_copy(zb, dacc.at[pl.ds(s * RPTP, RPTP)])
    plsc.subcore_barrier()

    base = w * EPW
    for b in range(RING):
        pltpu.async_copy(dst_hbm.at[pl.ds(base + b * CH, CH)],
                         didx.at[0, b], isx[b])

    def _round(r, carry):
        par = r % 2
        e0 = base + RING * r * CH
        for b in range(RING):
            @pl.when(r > 0)
            def _():
                pltpu.make_async_copy(
                    dst_hbm.at[pl.ds(0, CH)], didx.at[0, b], ssx[b]).wait()
            pltpu.make_async_copy(
                dst_hbm.at[pl.ds(e0 + b * CH, CH)],
                didx.at[par, b], isx[b]).wait()
            pltpu.async_copy(ones_v, dacc.at[didx.at[par, b]], ssx[b],
                             add=True)

            @pl.when(r < NROUND - 1)
            def _():
                pltpu.async_copy(
                    dst_hbm.at[pl.ds(e0 + (RING + b) * CH, CH)],
                    didx.at[1 - par, b], isx[b])
        return carry
    lax.fori_loop(0, NROUND, _round, 0)

    for b in range(RING):
        pltpu.make_async_copy(
            dst_hbm.at[pl.ds(0, CH)], didx.at[0, b], ssx[b]).wait()

    pltpu.sync_copy(dst_hbm.at[pl.ds(base + NROUND * RING * CH, CH)],
                    didx.at[0, 0])
    pltpu.sync_copy(ones_v, dacc.at[didx.at[0, 0]], add=True)

    plsc.subcore_barrier()
    pltpu.sync_copy(dacc.at[pl.ds(s * RPTP, RPTP)],
                    out_hbm.at[pl.ds(c * NP + s * RPTP, RPTP)])


@functools.lru_cache(maxsize=None)
def _get_sc_hist():
    return pl.kernel(
        _sc_hist_body,
        out_type=jax.ShapeDtypeStruct((NC * NP,), jnp.float32),
        mesh=plsc.VectorSubcoreMesh(core_axis_name="c", subcore_axis_name="s"),
        compiler_params=pltpu.CompilerParams(needs_layout_passes=False),
        scratch_types=[
            pltpu.VMEM((2, RING, CH), jnp.int32),
            pltpu.VMEM((CH,), jnp.float32),
            pltpu.VMEM((RPTP,), jnp.float32),
            pltpu.VMEM_SHARED((NP,), jnp.float32),
        ] + [pltpu.SemaphoreType.DMA] * (2 * RING),
    )


def _sc_hist(dst):
    return _get_sc_hist()(dst).reshape(NC, NP)


def _row_spec(shape=(RB, D)):
    return pl.BlockSpec(shape, lambda i: (i, 0))


def _full_spec(shape):
    return pl.BlockSpec(shape, lambda i: tuple(0 for _ in shape))


def _dot(a, b):
    return jnp.dot(a, b, preferred_element_type=jnp.float32)


def _tc_xw_body(x_ref, w_ref, xw_ref):
    xw_ref[...] = _dot(x_ref[...], w_ref[...])


def _tc_xw(x, gcn_W):
    return pl.pallas_call(
        _tc_xw_body,
        grid=(G,),
        in_specs=[_row_spec(), _full_spec((D, H))],
        out_specs=_row_spec(),
        out_shape=jax.ShapeDtypeStruct((N, H), jnp.float32),
    )(x, gcn_W)


def _tc_prep_body(xw_ref, deg_ref, y_ref, dinv_ref):
    dinv = lax.rsqrt(deg_ref[...])
    y_ref[...] = xw_ref[...] * dinv
    dinv_ref[...] = dinv


def _tc_prep(xw, deg2d):
    return pl.pallas_call(
        _tc_prep_body,
        grid=(G,),
        in_specs=[_row_spec(), _row_spec((RB, 1))],
        out_specs=[_row_spec(), _row_spec((RB, 1))],
        out_shape=[jax.ShapeDtypeStruct((N, H), jnp.float32),
                   jax.ShapeDtypeStruct((N, 1), jnp.float32)],
    )(xw, deg2d)


def _mp_specs():
    return [pl.BlockSpec((1, RB, H), lambda i: (0, i, 0)),
            pl.BlockSpec((1, RB, H), lambda i: (1, i, 0))]


def _tc_gcnpost_body(m0_ref, m1_ref, y_ref, dinv_ref, b_ref, w0_ref,
                     h_ref, hw_ref):
    h = dinv_ref[...] * (m0_ref[0] + m1_ref[0] + y_ref[...]) + b_ref[...]
    h_ref[...] = h
    hw_ref[...] = _dot(h, w0_ref[...])


def _tc_gcnpost(mp, y, dinv2d, gcn_b, W0):
    return pl.pallas_call(
        _tc_gcnpost_body,
        grid=(G,),
        in_specs=_mp_specs() + [
            _row_spec(), _row_spec((RB, 1)),
            _full_spec((1, H)), _full_spec((H, H)),
        ],
        out_specs=[_row_spec(), _row_spec()],
        out_shape=[jax.ShapeDtypeStruct((N, H), jnp.float32),
                   jax.ShapeDtypeStruct((N, H), jnp.float32)],
    )(mp, mp, y, dinv2d, gcn_b, W0)


def _gru_update(m, h, wih_ref, whh_ref, bih_ref, bhh_ref):
    gi = _dot(m, wih_ref[...]) + bih_ref[...]
    gh = _dot(h, whh_ref[...]) + bhh_ref[...]
    r = jax.nn.sigmoid(gi[:, :H] + gh[:, :H])
    z = jax.nn.sigmoid(gi[:, H:2 * H] + gh[:, H:2 * H])
    n = jnp.tanh(gi[:, 2 * H:] + r * gh[:, 2 * H:])
    return (1.0 - z) * n + z * h


def _tc_gru_mm_body(m0_ref, m1_ref, h_ref, wih_ref, whh_ref, bih_ref,
                    bhh_ref, wn_ref, hout_ref, hw_ref):
    hnew = _gru_update(m0_ref[0] + m1_ref[0], h_ref[...],
                       wih_ref, whh_ref, bih_ref, bhh_ref)
    hout_ref[...] = hnew
    hw_ref[...] = _dot(hnew, wn_ref[...])


def _tc_gru_mm(mp, h, WihT, WhhT, bih, bhh, Wnext):
    return pl.pallas_call(
        _tc_gru_mm_body,
        grid=(G,),
        in_specs=_mp_specs() + [
            _row_spec(),
            _full_spec((H, 3 * H)), _full_spec((H, 3 * H)),
            _full_spec((1, 3 * H)), _full_spec((1, 3 * H)),
            _full_spec((H, H)),
        ],
        out_specs=[_row_spec(), _row_spec()],
        out_shape=[jax.ShapeDtypeStruct((N, H), jnp.float32),
                   jax.ShapeDtypeStruct((N, H), jnp.float32)],
    )(mp, mp, h, WihT, WhhT, bih, bhh, Wnext)


def _tc_gru_head_body(m0_ref, m1_ref, h_ref, wih_ref, whh_ref, bih_ref,
                      bhh_ref, w1_ref, b1_ref, w2_ref, b2_ref, p_ref):
    hnew = _gru_update(m0_ref[0] + m1_ref[0], h_ref[...],
                       wih_ref, whh_ref, bih_ref, bhh_ref)
    o = jax.nn.relu(_dot(hnew, w1_ref[...]) + b1_ref[...])
    p_ref[...] = jax.nn.sigmoid(_dot(o, w2_ref[...]) + b2_ref[...])


def _tc_gru_head(mp, h, WihT, WhhT, bih, bhh, W1T, b1, W2T, b2):
    return pl.pallas_call(
        _tc_gru_head_body,
        grid=(G,),
        in_specs=_mp_specs() + [
            _row_spec(),
            _full_spec((H, 3 * H)), _full_spec((H, 3 * H)),
            _full_spec((1, 3 * H)), _full_spec((1, 3 * H)),
            _full_spec((H, H)), _full_spec((1, H)),
            _full_spec((H, 1)), _full_spec((1, 1)),
        ],
        out_specs=[_row_spec((RB, 1))],
        out_shape=[jax.ShapeDtypeStruct((N, 1), jnp.float32)],
    )(mp, mp, h, WihT, WhhT, bih, bhh, W1T, b1, W2T, b2)


def kernel(x, edge_index, gcn_W, gcn_b, ggc_W, gru_Wih, gru_Whh,
           gru_bih, gru_bhh, lin1_W, lin1_b, lin2_W, lin2_b):
    ei = edge_index.astype(jnp.int32)
    src = ei[0]
    dst = ei[1]
    pidx = jnp.stack([src.reshape(E // CH, CH), dst.reshape(E // CH, CH)],
                     axis=1)

    xw = _tc_xw(x, gcn_W)
    hist = _sc_hist(dst)
    deg2d = hist.sum(axis=0)[:N].reshape(N, 1) + 1.0

    zrows = jnp.zeros((RPTP, D), jnp.float32)
    y, dinv2d = _tc_prep(xw, deg2d)
    mp = _sc_scatter(y, pidx, zrows)
    h, hw = _tc_gcnpost(mp, y, dinv2d, gcn_b.reshape(1, H), ggc_W[0])

    WihT = gru_Wih.T
    WhhT = gru_Whh.T
    bih = gru_bih.reshape(1, 3 * H)
    bhh = gru_bhh.reshape(1, 3 * H)

    for i in range(L):
        mp = _sc_scatter(hw, pidx, zrows)
        if i < L - 1:
            h, hw = _tc_gru_mm(mp, h, WihT, WhhT, bih, bhh, ggc_W[i + 1])
        else:
            p, = _tc_gru_head(mp, h, WihT, WhhT, bih, bhh,
                              lin1_W.T, lin1_b.reshape(1, H),
                              lin2_W.T, lin2_b.reshape(1, 1))
    return p.reshape(N)


# speedup vs baseline: 1.0114x; 1.0114x over previous
import functools

import jax
import jax.numpy as jnp
from jax import lax
from jax.experimental import pallas as pl
from jax.experimental.pallas import tpu as pltpu
from jax.experimental.pallas import tpu_sc as plsc

N = 10000
D = 128
H = 128
E = 320000
L = 3

NC = 2
NS = 16
NW = NC * NS
EPW = E // NW
CH = 80
NCHUNK = EPW // CH
RING = 4
NROUND = NCHUNK // RING
RPT = N // NS
NP = 10240
RPTP = NP // NS
ZR = 128
HB = 640
CHI = 2000
EPC = E // NC

RB = 2000
G = N // RB


def _sc_scatter_body(y_hbm, idx_hbm, z_hbm, out_hbm, ib, rows, acc, *sems):
    gsx = sems[0:RING]
    ssx = sems[RING:2 * RING]
    isx = sems[2 * RING:3 * RING]
    c = lax.axis_index("c")
    s = lax.axis_index("s")
    w = c * NS + s

    pltpu.sync_copy(z_hbm, acc.at[pl.ds(s * RPTP, RPTP)])
    plsc.subcore_barrier()

    base = w * NCHUNK

    for b in range(RING):
        pltpu.async_copy(idx_hbm.at[base + b], ib.at[0, b], isx[b])

    def _round(r, carry):
        par = r % 2
        q0 = base + RING * r
        for b in range(RING):
            @pl.when(r > 0)
            def _():
                pltpu.make_async_copy(
                    y_hbm.at[pl.ds(0, CH)], rows.at[b], ssx[b]).wait()
            pltpu.make_async_copy(
                idx_hbm.at[q0 + b], ib.at[par, b], isx[b]).wait()
            pltpu.async_copy(y_hbm.at[ib.at[par, b, 0]], rows.at[b], gsx[b])
        for b in range(RING):
            pltpu.make_async_copy(
                y_hbm.at[pl.ds(0, CH)], rows.at[b], gsx[b]).wait()
            pltpu.async_copy(rows.at[b], acc.at[ib.at[par, b, 1]], ssx[b],
                             add=True)

            @pl.when(r < NROUND - 1)
            def _():
                pltpu.async_copy(idx_hbm.at[q0 + RING + b],
                                 ib.at[1 - par, b], isx[b])
        return carry
    lax.fori_loop(0, NROUND, _round, 0)

    for b in range(RING):
        pltpu.make_async_copy(
            y_hbm.at[pl.ds(0, CH)], rows.at[b], ssx[b]).wait()

    pltpu.sync_copy(idx_hbm.at[base + NROUND * RING], ib.at[0, 0])
    pltpu.async_copy(y_hbm.at[ib.at[0, 0, 0]], rows.at[0], gsx[0]).wait()
    pltpu.sync_copy(rows.at[0], acc.at[ib.at[0, 0, 1]], add=True)

    plsc.subcore_barrier()
    pltpu.sync_copy(acc.at[pl.ds(s * RPTP, RPTP)],
                    out_hbm.at[pl.ds(c * NP + s * RPTP, RPTP)])


@functools.lru_cache(maxsize=None)
def _get_sc_scatter():
    return pl.kernel(
        _sc_scatter_body,
        out_type=jax.ShapeDtypeStruct((NC * NP, D), jnp.float32),
        mesh=plsc.VectorSubcoreMesh(core_axis_name="c", subcore_axis_name="s"),
        compiler_params=pltpu.CompilerParams(needs_layout_passes=False),
        scratch_types=[
            pltpu.VMEM((2, RING, 2, CH), jnp.int32),
            pltpu.VMEM((RING, CH, D), jnp.float32),
            pltpu.VMEM_SHARED((NP, D), jnp.float32),
        ] + [pltpu.SemaphoreType.DMA] * (3 * RING),
    )


def _sc_scatter(y, pidx, zrows):
    return _get_sc_scatter()(y, pidx, zrows).reshape(NC, NP, D)


def _sc_hist_body(dst_hbm, out_hbm, didx, ones_v, zb, dacc, *sems):
    ssx = sems[0:RING]
    isx = sems[RING:2 * RING]
    c = lax.axis_index("c")
    s = lax.axis_index("s")
    w = c * NS + s

    def _fill(i, carry):
        ones_v[pl.ds(i * 16, 16)] = jnp.ones((16,), jnp.float32)
        return carry
    lax.fori_loop(0, CH // 16, _fill, 0)

    def _zb(i, carry):
        zb[pl.ds(i * 16, 16)] = jnp.zeros((16,), jnp.float32)
        return carry
    lax.fori_loop(0, RPTP // 16, _zb, 0)
    pltpu.sync_copy(zb, dacc.at[pl.ds(s * RPTP, RPTP)])
    plsc.subcore_barrier()

    base = w * EPW
    for b in range(RING):
        pltpu.async_copy(dst_hbm.at[pl.ds(base + b * CH, CH)],
                         didx.at[0, b], isx[b])

    def _round(r, carry):
        par = r % 2
        e0 = base + RING * r * CH
        for b in range(RING):
            @pl.when(r > 0)
            def _():
                pltpu.make_async_copy(
                    dst_hbm.at[pl.ds(0, CH)], didx.at[0, b], ssx[b]).wait()
            pltpu.make_async_copy(
                dst_hbm.at[pl.ds(e0 + b * CH, CH)],
                didx.at[par, b], isx[b]).wait()
            pltpu.async_copy(ones_v, dacc.at[didx.at[par, b]], ssx[b],
                             add=True)

            @pl.when(r < NROUND - 1)
            def _():
                pltpu.async_copy(
                    dst_hbm.at[pl.ds(e0 + (RING + b) * CH, CH)],
                    didx.at[1 - par, b], isx[b])
        return carry
    lax.fori_loop(0, NROUND, _round, 0)

    for b in range(RING):
        pltpu.make_async_copy(
            dst_hbm.at[pl.ds(0, CH)], didx.at[0, b], ssx[b]).wait()

    pltpu.sync_copy(dst_hbm.at[pl.ds(base + NROUND * RING * CH, CH)],
                    didx.at[0, 0])
    pltpu.sync_copy(ones_v, dacc.at[didx.at[0, 0]], add=True)

    plsc.subcore_barrier()
    pltpu.sync_copy(dacc.at[pl.ds(s * RPTP, RPTP)],
                    out_hbm.at[pl.ds(c * NP + s * RPTP, RPTP)])


@functools.lru_cache(maxsize=None)
def _get_sc_hist():
    return pl.kernel(
        _sc_hist_body,
        out_type=jax.ShapeDtypeStruct((NC * NP,), jnp.float32),
        mesh=plsc.VectorSubcoreMesh(core_axis_name="c", subcore_axis_name="s"),
        compiler_params=pltpu.CompilerParams(needs_layout_passes=False),
        scratch_types=[
            pltpu.VMEM((2, RING, CH), jnp.int32),
            pltpu.VMEM((CH,), jnp.float32),
            pltpu.VMEM((RPTP,), jnp.float32),
            pltpu.VMEM_SHARED((NP,), jnp.float32),
        ] + [pltpu.SemaphoreType.DMA] * (2 * RING),
    )


def _sc_hist(dst):
    return _get_sc_hist()(dst).reshape(NC, NP)


def _row_spec(shape=(RB, D)):
    return pl.BlockSpec(shape, lambda i: (i, 0))


def _full_spec(shape):
    return pl.BlockSpec(shape, lambda i: tuple(0 for _ in shape))


def _dot(a, b):
    return jnp.dot(a, b, preferred_element_type=jnp.float32)


def _tc_prep_body(x_ref, w_ref, deg_ref, y_ref, dinv_ref):
    dinv = lax.rsqrt(deg_ref[...])
    y_ref[...] = _dot(x_ref[...], w_ref[...]) * dinv
    dinv_ref[...] = dinv


def _tc_prep(x, gcn_W, deg2d):
    return pl.pallas_call(
        _tc_prep_body,
        grid=(G,),
        in_specs=[_row_spec(), _full_spec((D, H)), _row_spec((RB, 1))],
        out_specs=[_row_spec(), _row_spec((RB, 1))],
        out_shape=[jax.ShapeDtypeStruct((N, H), jnp.float32),
                   jax.ShapeDtypeStruct((N, 1), jnp.float32)],
    )(x, gcn_W, deg2d)


def _mp_specs():
    return [pl.BlockSpec((1, RB, H), lambda i: (0, i, 0)),
            pl.BlockSpec((1, RB, H), lambda i: (1, i, 0))]


def _tc_gcnpost_body(m0_ref, m1_ref, y_ref, dinv_ref, b_ref, w0_ref,
                     h_ref, hw_ref):
    h = dinv_ref[...] * (m0_ref[0] + m1_ref[0] + y_ref[...]) + b_ref[...]
    h_ref[...] = h
    hw_ref[...] = _dot(h, w0_ref[...])


def _tc_gcnpost(mp, y, dinv2d, gcn_b, W0):
    return pl.pallas_call(
        _tc_gcnpost_body,
        grid=(G,),
        in_specs=_mp_specs() + [
            _row_spec(), _row_spec((RB, 1)),
            _full_spec((1, H)), _full_spec((H, H)),
        ],
        out_specs=[_row_spec(), _row_spec()],
        out_shape=[jax.ShapeDtypeStruct((N, H), jnp.float32),
                   jax.ShapeDtypeStruct((N, H), jnp.float32)],
    )(mp, mp, y, dinv2d, gcn_b, W0)


def _gru_update(m, h, wih_ref, whh_ref, bih_ref, bhh_ref):
    gi = _dot(m, wih_ref[...]) + bih_ref[...]
    gh = _dot(h, whh_ref[...]) + bhh_ref[...]
    r = jax.nn.sigmoid(gi[:, :H] + gh[:, :H])
    z = jax.nn.sigmoid(gi[:, H:2 * H] + gh[:, H:2 * H])
    n = jnp.tanh(gi[:, 2 * H:] + r * gh[:, 2 * H:])
    return (1.0 - z) * n + z * h


def _tc_gru_mm_body(m0_ref, m1_ref, h_ref, wih_ref, whh_ref, bih_ref,
                    bhh_ref, wn_ref, hout_ref, hw_ref):
    hnew = _gru_update(m0_ref[0] + m1_ref[0], h_ref[...],
                       wih_ref, whh_ref, bih_ref, bhh_ref)
    hout_ref[...] = hnew
    hw_ref[...] = _dot(hnew, wn_ref[...])


def _tc_gru_mm(mp, h, WihT, WhhT, bih, bhh, Wnext):
    return pl.pallas_call(
        _tc_gru_mm_body,
        grid=(G,),
        in_specs=_mp_specs() + [
            _row_spec(),
            _full_spec((H, 3 * H)), _full_spec((H, 3 * H)),
            _full_spec((1, 3 * H)), _full_spec((1, 3 * H)),
            _full_spec((H, H)),
        ],
        out_specs=[_row_spec(), _row_spec()],
        out_shape=[jax.ShapeDtypeStruct((N, H), jnp.float32),
                   jax.ShapeDtypeStruct((N, H), jnp.float32)],
    )(mp, mp, h, WihT, WhhT, bih, bhh, Wnext)


def _tc_gru_head_body(m0_ref, m1_ref, h_ref, wih_ref, whh_ref, bih_ref,
                      bhh_ref, w1_ref, b1_ref, w2_ref, b2_ref, p_ref):
    hnew = _gru_update(m0_ref[0] + m1_ref[0], h_ref[...],
                       wih_ref, whh_ref, bih_ref, bhh_ref)
    o = jax.nn.relu(_dot(hnew, w1_ref[...]) + b1_ref[...])
    p_ref[...] = jax.nn.sigmoid(_dot(o, w2_ref[...]) + b2_ref[...])


def _tc_gru_head(mp, h, WihT, WhhT, bih, bhh, W1T, b1, W2T, b2):
    return pl.pallas_call(
        _tc_gru_head_body,
        grid=(G,),
        in_specs=_mp_specs() + [
            _row_spec(),
            _full_spec((H, 3 * H)), _full_spec((H, 3 * H)),
            _full_spec((1, 3 * H)), _full_spec((1, 3 * H)),
            _full_spec((H, H)), _full_spec((1, H)),
            _full_spec((H, 1)), _full_spec((1, 1)),
        ],
        out_specs=[_row_spec((RB, 1))],
        out_shape=[jax.ShapeDtypeStruct((N, 1), jnp.float32)],
    )(mp, mp, h, WihT, WhhT, bih, bhh, W1T, b1, W2T, b2)


def kernel(x, edge_index, gcn_W, gcn_b, ggc_W, gru_Wih, gru_Whh,
           gru_bih, gru_bhh, lin1_W, lin1_b, lin2_W, lin2_b):
    ei = edge_index.astype(jnp.int32)
    src = ei[0]
    dst = ei[1]
    pidx = jnp.stack([src.reshape(E // CH, CH), dst.reshape(E // CH, CH)],
                     axis=1)

    hist = _sc_hist(dst)
    deg2d = hist.sum(axis=0)[:N].reshape(N, 1) + 1.0

    zrows = jnp.zeros((RPTP, D), jnp.float32)
    y, dinv2d = _tc_prep(x, gcn_W, deg2d)
    mp = _sc_scatter(y, pidx, zrows)
    h, hw = _tc_gcnpost(mp, y, dinv2d, gcn_b.reshape(1, H), ggc_W[0])

    WihT = gru_Wih.T
    WhhT = gru_Whh.T
    bih = gru_bih.reshape(1, 3 * H)
    bhh = gru_bhh.reshape(1, 3 * H)

    for i in range(L):
        mp = _sc_scatter(hw, pidx, zrows)
        if i < L - 1:
            h, hw = _tc_gru_mm(mp, h, WihT, WhhT, bih, bhh, ggc_W[i + 1])
        else:
            p, = _tc_gru_head(mp, h, WihT, WhhT, bih, bhh,
                              lin1_W.T, lin1_b.reshape(1, H),
                              lin2_W.T, lin2_b.reshape(1, 1))
    return p.reshape(N)
